# baseline (speedup 1.0000x reference)
import math

import jax
import jax.numpy as jnp
from jax import lax
from jax.experimental import pallas as pl
from jax.experimental.pallas import tpu as pltpu
from jax.experimental.pallas import tpu_sc as plsc

N = 32768
B = 16
NCV = 128
D = 6
NF = 8
EPS = 1e-10
_HALF_LOG_2PI = 0.5 * math.log(2.0 * math.pi)
_LN2 = 0.6931471805599453

NTILES = 32
PTS = N // NTILES
XCH = PTS * D


def _prep_body(scale_ref, logit_ref, out_ref):
    s = jnp.maximum(scale_ref[...], EPS)
    out_ref[0] = 1.0 / (s * s)
    out_ref[1] = logit_ref[...] - D * jnp.log(s) - D * _HALF_LOG_2PI


def _ln16(v):
    iv = lax.bitcast_convert_type(v, jnp.int32)
    e = lax.shift_right_logical(iv, 23) - 127
    mant = lax.bitcast_convert_type(
        jnp.bitwise_or(jnp.bitwise_and(iv, 0x007FFFFF), 0x3F800000),
        jnp.float32)
    big = mant >= 1.4142135623730951
    m2 = jnp.where(big, mant * 0.5, mant)
    e2 = jnp.where(big, e + 1, e).astype(jnp.float32)
    s = (m2 - 1.0) / (m2 + 1.0)
    s2 = s * s
    p = s * (2.0 + s2 * (2.0 / 3.0 + s2 * (2.0 / 5.0 + s2 * (2.0 / 7.0 + s2 * (2.0 / 9.0)))))
    return e2 * _LN2 + p


_GATHER_DNUMS = lax.GatherDimensionNumbers(
    offset_dims=(), collapsed_slice_dims=(0,), start_index_map=(0,))


def _shuf(v, sh):
    idx = jnp.bitwise_xor(lax.iota(jnp.int32, 16), sh)
    return lax.gather(v, idx[:, None], _GATHER_DNUMS, (1,),
                      mode=lax.GatherScatterMode.PROMISE_IN_BOUNDS)


def _allmax(v):
    for sh in (8, 4, 2, 1):
        v = jnp.maximum(v, _shuf(v, sh))
    return v


def _allsum(v):
    for sh in (8, 4, 2, 1):
        v = v + _shuf(v, sh)
    return v


def _sc_body(votes_hbm, prep_hbm, x_hbm, batch_hbm, out_hbm,
             votes_v, prep_v, x_v, batch_v, seg_v):
    wid = lax.axis_index("s") * 2 + lax.axis_index("c")

    pltpu.sync_copy(votes_hbm, votes_v)
    pltpu.sync_copy(prep_hbm, prep_v)
    pltpu.sync_copy(x_hbm.at[pl.ds(wid * XCH, XCH)], x_v)
    pltpu.sync_copy(batch_hbm.at[pl.ds(wid * PTS, PTS)], batch_v)

    lanes = lax.iota(jnp.int32, 16)

    def group16(p16, seg_acc):
        b16 = batch_v[pl.ds(p16 * 16, 16)]
        xv = [x_v[pl.ds(p16 * (16 * D) + k * 16, 16)] for k in range(D)]
        for j in range(16):
            b = b16[j]
            xs = [xv[(j * D + d) // 16][(j * D + d) % 16] for d in range(D)]
            posts = []
            for g in range(NF):
                off = g * 16
                acc = jnp.zeros((16,), jnp.float32)
                for d in range(D):
                    m = votes_v[d, b, pl.ds(off, 16)]
                    diff = jnp.full((16,), xs[d]) - m
                    acc = acc + diff * diff
                inv2 = prep_v[0, b, pl.ds(off, 16)]
                cst = prep_v[1, b, pl.ds(off, 16)]
                posts.append(cst - 0.5 * acc * inv2)
            mv = posts[0]
            for g in range(1, NF):
                mv = jnp.maximum(mv, posts[g])
            mfull = _allmax(mv)
            sv = jnp.zeros((16,), jnp.float32)
            for g in range(NF):
                sv = sv + jnp.exp(posts[g] - mfull)
            lpp = mfull + _ln16(_allsum(sv))
            seg_acc = seg_acc + jnp.where(lanes == jnp.full((16,), b),
                                          lpp, 0.0)
        return seg_acc

    seg = lax.fori_loop(0, PTS // 16, group16, jnp.zeros((16,), jnp.float32))
    seg_v[...] = seg
    pltpu.sync_copy(seg_v, out_hbm.at[wid])


def _finish_body(parts_ref, seg_ref, mean_ref):
    seg = jnp.sum(parts_ref[...], axis=0)
    seg_ref[...] = seg[None, :]
    mean_ref[...] = jnp.sum(seg, keepdims=True)[None, :] * (1.0 / B)


@jax.jit
def kernel(x, vote_6d, scale, vote_presence_logit, batch):
    votes_t = jnp.transpose(vote_6d.reshape(B, NCV, D), (2, 0, 1))
    scale_r = scale.reshape(B, NCV)
    logit_r = vote_presence_logit.reshape(B, NCV)

    prep = pl.pallas_call(
        _prep_body,
        out_shape=jax.ShapeDtypeStruct((2, B, NCV), jnp.float32),
    )(scale_r, logit_r)

    mesh = plsc.VectorSubcoreMesh(core_axis_name="c", subcore_axis_name="s")
    sc = pl.kernel(
        _sc_body,
        out_type=jax.ShapeDtypeStruct((NTILES, B), jnp.float32),
        mesh=mesh,
        scratch_types=[
            pltpu.VMEM((D, B, NCV), jnp.float32),
            pltpu.VMEM((2, B, NCV), jnp.float32),
            pltpu.VMEM((XCH,), jnp.float32),
            pltpu.VMEM((PTS,), jnp.int32),
            pltpu.VMEM((B,), jnp.float32),
        ],
    )
    parts = sc(votes_t, prep, x.reshape(N * D), batch)

    seg2d, mean2d = pl.pallas_call(
        _finish_body,
        out_shape=[
            jax.ShapeDtypeStruct((1, B), jnp.float32),
            jax.ShapeDtypeStruct((1, 1), jnp.float32),
        ],
    )(parts)
    return (mean2d.reshape(()), seg2d.reshape(B))


# speedup vs baseline: 2.2363x; 2.2363x over previous
import math

import jax
import jax.numpy as jnp
from jax import lax
from jax.experimental import pallas as pl
from jax.experimental.pallas import tpu as pltpu
from jax.experimental.pallas import tpu_sc as plsc

N = 32768
B = 16
NCV = 128
D = 6
NF = 8
EPS = 1e-10
_HALF_LOG_2PI = 0.5 * math.log(2.0 * math.pi)
_LN2 = 0.6931471805599453

NTILES = 32
N_SC = 4096
N_TC = N - N_SC
BLK = 4096
GRID = N_TC // BLK
PTS = N_SC // NTILES
XCH = PTS * D


def _tc_body(x_ref, votes_ref, scale_ref, logit_ref, batch_ref, seg_ref,
             w_ref):
    i = pl.program_id(0)

    @pl.when(i == 0)
    def _():
        s = jnp.maximum(scale_ref[...], EPS)
        inv2 = 1.0 / (s * s)
        msq = jnp.zeros((B, NCV), jnp.float32)
        for d in range(D):
            msq = msq + votes_ref[d] * votes_ref[d]
        const_row = (logit_ref[...] - D * jnp.log(s) - D * _HALF_LOG_2PI
                     - 0.5 * msq * inv2)
        w_ref[...] = jnp.concatenate(
            [votes_ref[d] * inv2 for d in range(D)]
            + [-0.5 * inv2, const_row], axis=0)
        seg_ref[...] = jnp.zeros_like(seg_ref)

    x = x_ref[...]
    xsq = x * x

    ej = lax.broadcasted_iota(jnp.int32, (D, NF * B), 0)
    el = lax.broadcasted_iota(jnp.int32, (D, NF * B), 1)
    ea = jnp.where(ej == el // B, 1.0, 0.0).astype(jnp.float32)
    eb = jnp.where(el // B == D, 1.0, 0.0).astype(jnp.float32)
    xe = (lax.dot(x, ea, preferred_element_type=jnp.float32)
          + lax.dot(xsq, eb, preferred_element_type=jnp.float32))

    lanes = lax.broadcasted_iota(jnp.int32, (BLK, NF * B), 1)
    xe = jnp.where(lanes // B == NF - 1, 1.0, xe)

    bids = batch_ref[...]
    f = jnp.where(bids == lanes % B, xe, 0.0)
    post = lax.dot(f, w_ref[...],
                   preferred_element_type=jnp.float32)

    mx = jnp.max(post, axis=1, keepdims=True)
    sexp = lax.dot(jnp.exp(post - mx), jnp.ones((NCV, 1), jnp.float32),
                   preferred_element_type=jnp.float32)
    lpp = mx + jnp.log(sexp)

    cols16 = lax.broadcasted_iota(jnp.int32, (BLK, B), 1)
    oh16 = jnp.where(bids == cols16, lpp, 0.0)
    seg_ref[...] += jnp.sum(oh16, axis=0)[None, :]


def _prep_body(scale_ref, logit_ref, out_ref):
    s = jnp.maximum(scale_ref[...], EPS)
    out_ref[0] = 1.0 / (s * s)
    out_ref[1] = logit_ref[...] - D * jnp.log(s) - D * _HALF_LOG_2PI


def _ln16(v):
    iv = lax.bitcast_convert_type(v, jnp.int32)
    e = lax.shift_right_logical(iv, 23) - 127
    mant = lax.bitcast_convert_type(
        jnp.bitwise_or(jnp.bitwise_and(iv, 0x007FFFFF), 0x3F800000),
        jnp.float32)
    big = mant >= 1.4142135623730951
    m2 = jnp.where(big, mant * 0.5, mant)
    e2 = jnp.where(big, e + 1, e).astype(jnp.float32)
    s = (m2 - 1.0) / (m2 + 1.0)
    s2 = s * s
    p = s * (2.0 + s2 * (2.0 / 3.0 + s2 * (2.0 / 5.0
             + s2 * (2.0 / 7.0 + s2 * (2.0 / 9.0)))))
    return e2 * _LN2 + p


_GATHER_DNUMS = lax.GatherDimensionNumbers(
    offset_dims=(), collapsed_slice_dims=(0,), start_index_map=(0,))


def _shuf(v, sh):
    idx = jnp.bitwise_xor(lax.iota(jnp.int32, 16), sh)
    return lax.gather(v, idx[:, None], _GATHER_DNUMS, (1,),
                      mode=lax.GatherScatterMode.PROMISE_IN_BOUNDS)


def _allmax(v):
    for sh in (8, 4, 2, 1):
        v = jnp.maximum(v, _shuf(v, sh))
    return v


def _allsum(v):
    for sh in (8, 4, 2, 1):
        v = v + _shuf(v, sh)
    return v


def _sc_body(votes_hbm, prep_hbm, x_hbm, batch_hbm, out_hbm,
             votes_v, prep_v, x_v, batch_v, seg_v):
    wid = lax.axis_index("s") * 2 + lax.axis_index("c")

    pltpu.sync_copy(votes_hbm, votes_v)
    pltpu.sync_copy(prep_hbm, prep_v)
    pltpu.sync_copy(x_hbm.at[pl.ds(N_TC * D + wid * XCH, XCH)], x_v)
    pltpu.sync_copy(batch_hbm.at[pl.ds(N_TC + wid * PTS, PTS)], batch_v)

    lanes = lax.iota(jnp.int32, 16)

    def group16(p16, seg_acc):
        b16 = batch_v[pl.ds(p16 * 16, 16)]
        xv = [x_v[pl.ds(p16 * (16 * D) + k * 16, 16)] for k in range(D)]
        for j in range(16):
            b = b16[j]
            xs = [xv[(j * D + d) // 16][(j * D + d) % 16] for d in range(D)]
            posts = []
            for g in range(NF):
                off = g * 16
                acc = jnp.zeros((16,), jnp.float32)
                for d in range(D):
                    m = votes_v[d, b, pl.ds(off, 16)]
                    diff = jnp.full((16,), xs[d]) - m
                    acc = acc + diff * diff
                inv2 = prep_v[0, b, pl.ds(off, 16)]
                cst = prep_v[1, b, pl.ds(off, 16)]
                posts.append(cst - 0.5 * acc * inv2)
            mv = posts[0]
            for g in range(1, NF):
                mv = jnp.maximum(mv, posts[g])
            mfull = _allmax(mv)
            sv = jnp.zeros((16,), jnp.float32)
            for g in range(NF):
                sv = sv + jnp.exp(posts[g] - mfull)
            lpp = mfull + _ln16(_allsum(sv))
            seg_acc = seg_acc + jnp.where(lanes == jnp.full((16,), b),
                                          lpp, 0.0)
        return seg_acc

    seg = lax.fori_loop(0, PTS // 16, group16,
                        jnp.zeros((16,), jnp.float32))
    seg_v[...] = seg
    pltpu.sync_copy(seg_v, out_hbm.at[wid])


def _finish_body(tcseg_ref, parts_ref, seg_ref, mean_ref):
    seg = tcseg_ref[0] + jnp.sum(parts_ref[...], axis=0)
    seg_ref[...] = seg[None, :]
    mean_ref[...] = jnp.sum(seg, keepdims=True)[None, :] * (1.0 / B)


@jax.jit
def kernel(x, vote_6d, scale, vote_presence_logit, batch):
    votes_t = jnp.transpose(vote_6d.reshape(B, NCV, D), (2, 0, 1))
    scale_r = scale.reshape(B, NCV)
    logit_r = vote_presence_logit.reshape(B, NCV)
    batch_c = batch.reshape(N, 1)

    prep = pl.pallas_call(
        _prep_body,
        out_shape=jax.ShapeDtypeStruct((2, B, NCV), jnp.float32),
    )(scale_r, logit_r)

    mesh = plsc.VectorSubcoreMesh(core_axis_name="c", subcore_axis_name="s")
    sc_parts = pl.kernel(
        _sc_body,
        out_type=jax.ShapeDtypeStruct((NTILES, B), jnp.float32),
        mesh=mesh,
        scratch_types=[
            pltpu.VMEM((D, B, NCV), jnp.float32),
            pltpu.VMEM((2, B, NCV), jnp.float32),
            pltpu.VMEM((XCH,), jnp.float32),
            pltpu.VMEM((PTS,), jnp.int32),
            pltpu.VMEM((B,), jnp.float32),
        ],
    )(votes_t, prep, x.reshape(N * D), batch)

    tc_seg = pl.pallas_call(
        _tc_body,
        grid=(GRID,),
        in_specs=[
            pl.BlockSpec((BLK, D), lambda i: (i, 0)),
            pl.BlockSpec((D, B, NCV), lambda i: (0, 0, 0)),
            pl.BlockSpec((B, NCV), lambda i: (0, 0)),
            pl.BlockSpec((B, NCV), lambda i: (0, 0)),
            pl.BlockSpec((BLK, 1), lambda i: (i, 0)),
        ],
        out_specs=pl.BlockSpec((1, B), lambda i: (0, 0)),
        out_shape=jax.ShapeDtypeStruct((1, B), jnp.float32),
        scratch_shapes=[pltpu.VMEM((NF * B, NCV), jnp.float32)],
        compiler_params=pltpu.CompilerParams(
            dimension_semantics=("arbitrary",)),
    )(x[:N_TC], votes_t, scale_r, logit_r, batch_c[:N_TC])

    seg2d, mean2d = pl.pallas_call(
        _finish_body,
        out_shape=[
            jax.ShapeDtypeStruct((1, B), jnp.float32),
            jax.ShapeDtypeStruct((1, 1), jnp.float32),
        ],
    )(tc_seg, sc_parts)
    return (mean2d.reshape(()), seg2d.reshape(B))


# speedup vs baseline: 4.0229x; 1.7989x over previous
import math

import jax
import jax.numpy as jnp
from jax.experimental import pallas as pl
from jax.experimental.pallas import tpu as pltpu

N = 32768
B = 16
NCV = 128
D = 6
NF = 8
EPS = 1e-10
BLK = 16384
GRID = N // BLK
_HALF_LOG_2PI = 0.5 * math.log(2.0 * math.pi)


def _body(x_ref, votes_ref, scale_ref, logit_ref, batch_ref, seg_ref, mean_ref,
          w_ref):
    i = pl.program_id(0)

    @pl.when(i == 0)
    def _():
        s = jnp.maximum(scale_ref[...], EPS)
        inv2 = 1.0 / (s * s)
        msq = jnp.zeros((B, NCV), jnp.float32)
        for d in range(D):
            msq = msq + votes_ref[d] * votes_ref[d]
        const_row = (logit_ref[...] - D * jnp.log(s) - D * _HALF_LOG_2PI
                     - 0.5 * msq * inv2)
        w_ref[...] = jnp.concatenate(
            [votes_ref[d] * inv2 for d in range(D)]
            + [-0.5 * inv2, const_row], axis=0)
        seg_ref[...] = jnp.zeros_like(seg_ref)

    x = x_ref[...]
    xsq = x * x

    ej = jax.lax.broadcasted_iota(jnp.int32, (D, NF * B), 0)
    el = jax.lax.broadcasted_iota(jnp.int32, (D, NF * B), 1)
    ea = jnp.where(ej == el // B, 1.0, 0.0).astype(jnp.float32)
    eb = jnp.where(el // B == D, 1.0, 0.0).astype(jnp.float32)
    xe = (jax.lax.dot(x, ea, preferred_element_type=jnp.float32)
          + jax.lax.dot(xsq, eb, preferred_element_type=jnp.float32))

    lanes = jax.lax.broadcasted_iota(jnp.int32, (BLK, NF * B), 1)
    xe = jnp.where(lanes // B == NF - 1, 1.0, xe)

    bids = batch_ref[...]
    f = jnp.where(bids == lanes % B, xe, 0.0)
    post = jax.lax.dot(f, w_ref[...],
                       preferred_element_type=jnp.float32)

    mx = jnp.max(post, axis=1, keepdims=True)
    sexp = jax.lax.dot(jnp.exp(post - mx), jnp.ones((NCV, 1), jnp.float32),
                       preferred_element_type=jnp.float32)
    lpp = mx + jnp.log(sexp)

    cols16 = jax.lax.broadcasted_iota(jnp.int32, (BLK, B), 1)
    oh16 = jnp.where(bids == cols16, lpp, 0.0)
    seg_ref[...] += jnp.sum(oh16, axis=0)[None, :]

    @pl.when(i == GRID - 1)
    def _():
        mean_ref[...] = jnp.sum(seg_ref[...], keepdims=True) * (1.0 / B)


@jax.jit
def kernel(x, vote_6d, scale, vote_presence_logit, batch):
    votes_t = jnp.transpose(vote_6d.reshape(B, NCV, D), (2, 0, 1))
    scale_r = scale.reshape(B, NCV)
    logit_r = vote_presence_logit.reshape(B, NCV)
    batch_c = batch.reshape(N, 1)

    seg2d, mean2d = pl.pallas_call(
        _body,
        grid=(GRID,),
        in_specs=[
            pl.BlockSpec((BLK, D), lambda i: (i, 0)),
            pl.BlockSpec((D, B, NCV), lambda i: (0, 0, 0)),
            pl.BlockSpec((B, NCV), lambda i: (0, 0)),
            pl.BlockSpec((B, NCV), lambda i: (0, 0)),
            pl.BlockSpec((BLK, 1), lambda i: (i, 0)),
        ],
        out_specs=[
            pl.BlockSpec((1, B), lambda i: (0, 0)),
            pl.BlockSpec((1, 1), lambda i: (0, 0)),
        ],
        out_shape=[
            jax.ShapeDtypeStruct((1, B), jnp.float32),
            jax.ShapeDtypeStruct((1, 1), jnp.float32),
        ],
        scratch_shapes=[pltpu.VMEM((NF * B, NCV), jnp.float32)],
        compiler_params=pltpu.CompilerParams(
            dimension_semantics=("arbitrary",)),
    )(x, votes_t, scale_r, logit_r, batch_c)
    return (mean2d.reshape(()), seg2d.reshape(B))
